# Initial kernel scaffold; baseline (speedup 1.0000x reference)
import jax
import jax.numpy as jnp
from jax.experimental import pallas as pl


def kernel(x, edge_index, edge_attr, batch, W_enc, b_enc, edge_table, W1, b1, ln1s, ln1b, W2, b2, norm_s, norm_b, W_pred, b_pred):
    raise NotImplementedError("write your pallas kernel here")


# speedup vs baseline: 1.0023x; 1.0023x over previous
import functools

import jax
import jax.numpy as jnp
from jax.experimental import pallas as pl

N = 10000
E = 320000
H = 128
L = 7
G = 64
VOCAB = 8
EPS = 1e-7

_BLK = 1000


def _enc_body(x_ref, w_ref, b_ref, out_ref):
    out_ref[...] = x_ref[...] @ w_ref[...] + b_ref[...]


def _encode(x, W_enc, b_enc):
    return pl.pallas_call(
        _enc_body,
        grid=(N // _BLK,),
        in_specs=[
            pl.BlockSpec((_BLK, H), lambda i: (i, 0)),
            pl.BlockSpec((H, H), lambda i: (0, 0)),
            pl.BlockSpec((1, H), lambda i: (0, 0)),
        ],
        out_specs=pl.BlockSpec((_BLK, H), lambda i: (i, 0)),
        out_shape=jax.ShapeDtypeStruct((N, H), jnp.float32),
    )(x, W_enc, b_enc.reshape(1, H))


def _mlp_body(h2_ref, m_ref, res_ref, w1_ref, b1_ref, s1_ref, bb1_ref,
              w2_ref, b2_ref, out_ref):
    hsum = h2_ref[...] + m_ref[...]
    t = hsum @ w1_ref[...] + b1_ref[...]
    mu = jnp.mean(t, axis=-1, keepdims=True)
    var = jnp.mean((t - mu) ** 2, axis=-1, keepdims=True)
    t = (t - mu) / jnp.sqrt(var + 1e-5) * s1_ref[...] + bb1_ref[...]
    t = jnp.maximum(t, 0.0)
    out_ref[...] = t @ w2_ref[...] + b2_ref[...] + res_ref[...]


def _mlp(h2, m, res, W1l, b1l, s1l, bb1l, W2l, b2l):
    return pl.pallas_call(
        _mlp_body,
        grid=(N // _BLK,),
        in_specs=[
            pl.BlockSpec((_BLK, H), lambda i: (i, 0)),
            pl.BlockSpec((_BLK, H), lambda i: (i, 0)),
            pl.BlockSpec((_BLK, H), lambda i: (i, 0)),
            pl.BlockSpec((H, 2 * H), lambda i: (0, 0)),
            pl.BlockSpec((1, 2 * H), lambda i: (0, 0)),
            pl.BlockSpec((1, 2 * H), lambda i: (0, 0)),
            pl.BlockSpec((1, 2 * H), lambda i: (0, 0)),
            pl.BlockSpec((2 * H, H), lambda i: (0, 0)),
            pl.BlockSpec((1, H), lambda i: (0, 0)),
        ],
        out_specs=pl.BlockSpec((_BLK, H), lambda i: (i, 0)),
        out_shape=jax.ShapeDtypeStruct((N, H), jnp.float32),
    )(h2, m, res, W1l, b1l.reshape(1, -1), s1l.reshape(1, -1),
      bb1l.reshape(1, -1), W2l, b2l.reshape(1, -1))


def _layernorm(x, s, b):
    mu = jnp.mean(x, axis=-1, keepdims=True)
    var = jnp.var(x, axis=-1, keepdims=True)
    return (x - mu) / jnp.sqrt(var + 1e-5) * s + b


def kernel(x, edge_index, edge_attr, batch, W_enc, b_enc, edge_table, W1, b1,
           ln1s, ln1b, W2, b2, norm_s, norm_b, W_pred, b_pred):
    src = edge_index[0]
    dst = edge_index[1]
    h = _encode(x, W_enc, b_enc)
    emb = edge_table[edge_attr]
    zeros = jnp.zeros((N, H), jnp.float32)
    for l in range(L):
        if l == 0:
            h2 = h
            res = zeros
        else:
            h2 = jax.nn.relu(_layernorm(h, norm_s[l - 1], norm_b[l - 1]))
            res = h
        msg = jax.nn.relu(h2[src] + emb) + EPS
        m = jax.ops.segment_sum(msg, dst, num_segments=N)
        h = _mlp(h2, m, res, W1[l], b1[l], ln1s[l], ln1b[l], W2[l], b2[l])
    hf = _layernorm(h, norm_s[L - 1], norm_b[L - 1])
    sums = jax.ops.segment_sum(hf, batch, num_segments=G)
    counts = jax.ops.segment_sum(jnp.ones((N,), jnp.float32), batch,
                                 num_segments=G)
    hg = sums / jnp.maximum(counts, 1.0)[:, None]
    out = jax.nn.sigmoid(hg @ W_pred + b_pred)
    return out.reshape(-1)


# speedup vs baseline: 3.1218x; 3.1145x over previous
import functools

import jax
import jax.numpy as jnp
from jax import lax
from jax.experimental import pallas as pl
from jax.experimental.pallas import tpu as pltpu
from jax.experimental.pallas import tpu_sc as plsc

N = 10000
E = 320000
H = 128
L = 7
G = 64
VOCAB = 8
EPS = 1e-7

_BLK = 1000

_NSC = 2
_NSUB = 16
_NW = _NSC * _NSUB
_CSZ = 128
_CH = 80
_NBUF = 2
_NGRP = _CH // _NBUF
_NGT = _NW * _NGRP + 1
_NPAD = 10112
_RPT = _NPAD // _NSUB


def _mp_body(xp_hbm, gidx_hbm, didx_hbm, zeros_hbm, out_hbm,
             m_sh, gib0, gib1, dib0, dib1, r0, r1,
             gs0, gs1, ss0, ss1, is0, is1):
    c = lax.axis_index("c")
    s = lax.axis_index("s")
    wid = c * _NSUB + s
    gbase = wid * _NGRP
    bufs = (r0, r1)
    gsems = (gs0, gs1)
    ssems = (ss0, ss1)
    gibs = (gib0, gib1)
    dibs = (dib0, dib1)
    isems = (is0, is1)

    def _idx_load(g, p):
        pltpu.async_copy(gidx_hbm.at[gbase + g], gibs[p], isems[p])
        pltpu.async_copy(didx_hbm.at[gbase + g], dibs[p], isems[p])

    def _idx_wait(p):
        pltpu.make_async_copy(gidx_hbm.at[0], gibs[p], isems[p]).wait()
        pltpu.make_async_copy(didx_hbm.at[0], dibs[p], isems[p]).wait()

    def _start_gather(p, b):
        pltpu.async_copy(xp_hbm.at[gibs[p].at[b]], bufs[b], gsems[b])

    def _wait_gather(p, b):
        pltpu.make_async_copy(xp_hbm.at[gibs[p].at[b]], bufs[b],
                              gsems[b]).wait()

    def _start_scatter(p, b):
        pltpu.async_copy(bufs[b], m_sh.at[dibs[p].at[b]], ssems[b], add=True)

    def _wait_scatter(p, b):
        pltpu.make_async_copy(bufs[b], m_sh.at[dibs[p].at[b]],
                              ssems[b]).wait()

    pltpu.sync_copy(zeros_hbm.at[pl.ds(s * _RPT, _RPT)],
                    m_sh.at[pl.ds(s * _RPT, _RPT)])

    _idx_load(0, 0)
    _idx_load(1, 1)
    plsc.subcore_barrier()
    _idx_wait(0)
    for b in range(_NBUF):
        _start_gather(0, b)

    def _do_group(p, q, prefetch_g):
        for b in range(_NBUF):
            _wait_gather(p, b)
            _start_scatter(p, b)
        _idx_wait(q)
        for b in range(_NBUF):
            _wait_scatter(p, b)
            _start_gather(q, b)
        _idx_load(prefetch_g, p)

    def _pair(i, carry):
        g = 2 * i
        _do_group(0, 1, g + 2)
        _do_group(1, 0, g + 3)
        return carry

    lax.fori_loop(0, _NGRP // 2 - 1, _pair, 0)
    _do_group(0, 1, _NGRP)
    for b in range(_NBUF):
        _wait_gather(1, b)
        _start_scatter(1, b)
    for b in range(_NBUF):
        _wait_scatter(1, b)
    _idx_wait(0)

    plsc.subcore_barrier()
    pltpu.sync_copy(m_sh.at[pl.ds(s * _RPT, _RPT)],
                    out_hbm.at[c, pl.ds(s * _RPT, _RPT)])


_mp_call = pl.kernel(
    _mp_body,
    out_type=jax.ShapeDtypeStruct((_NSC, _NPAD, H), jnp.float32),
    mesh=plsc.VectorSubcoreMesh(core_axis_name="c", subcore_axis_name="s"),
    scratch_types=[
        pltpu.VMEM_SHARED((_NPAD, H), jnp.float32),
        pltpu.VMEM((_NBUF, _CSZ), jnp.int32),
        pltpu.VMEM((_NBUF, _CSZ), jnp.int32),
        pltpu.VMEM((_NBUF, _CSZ), jnp.int32),
        pltpu.VMEM((_NBUF, _CSZ), jnp.int32),
        pltpu.VMEM((_CSZ, H), jnp.float32),
        pltpu.VMEM((_CSZ, H), jnp.float32),
        pltpu.SemaphoreType.DMA,
        pltpu.SemaphoreType.DMA,
        pltpu.SemaphoreType.DMA,
        pltpu.SemaphoreType.DMA,
        pltpu.SemaphoreType.DMA,
        pltpu.SemaphoreType.DMA,
    ],
)


def _enc_body(x_ref, w_ref, b_ref, out_ref):
    out_ref[...] = x_ref[...] @ w_ref[...] + b_ref[...]


def _encode(x, W_enc, b_enc):
    return pl.pallas_call(
        _enc_body,
        grid=(N // _BLK,),
        in_specs=[
            pl.BlockSpec((_BLK, H), lambda i: (i, 0)),
            pl.BlockSpec((H, H), lambda i: (0, 0)),
            pl.BlockSpec((1, H), lambda i: (0, 0)),
        ],
        out_specs=pl.BlockSpec((_BLK, H), lambda i: (i, 0)),
        out_shape=jax.ShapeDtypeStruct((N, H), jnp.float32),
    )(x, W_enc, b_enc.reshape(1, H))


def _xp_body(h2_ref, table_ref, out_ref):
    h2 = h2_ref[...]
    for a in range(VOCAB):
        out_ref[a] = jnp.maximum(h2 + table_ref[a], 0.0) + EPS


def _xprime(h2, table):
    out = pl.pallas_call(
        _xp_body,
        grid=(N // _BLK,),
        in_specs=[
            pl.BlockSpec((_BLK, H), lambda i: (i, 0)),
            pl.BlockSpec((VOCAB, H), lambda i: (0, 0)),
        ],
        out_specs=pl.BlockSpec((VOCAB, _BLK, H), lambda i: (0, i, 0)),
        out_shape=jax.ShapeDtypeStruct((VOCAB, N, H), jnp.float32),
    )(h2, table)
    return out.reshape(VOCAB * N, H)


def _mlp_body(h2_ref, parts_ref, res_ref, w1_ref, b1_ref, s1_ref, bb1_ref,
              w2_ref, b2_ref, out_ref):
    hsum = h2_ref[...] + parts_ref[0] + parts_ref[1]
    t = hsum @ w1_ref[...] + b1_ref[...]
    mu = jnp.mean(t, axis=-1, keepdims=True)
    var = jnp.mean((t - mu) ** 2, axis=-1, keepdims=True)
    t = (t - mu) / jnp.sqrt(var + 1e-5) * s1_ref[...] + bb1_ref[...]
    t = jnp.maximum(t, 0.0)
    out_ref[...] = t @ w2_ref[...] + b2_ref[...] + res_ref[...]


def _mlp(h2, parts, res, W1l, b1l, s1l, bb1l, W2l, b2l):
    return pl.pallas_call(
        _mlp_body,
        grid=(N // _BLK,),
        in_specs=[
            pl.BlockSpec((_BLK, H), lambda i: (i, 0)),
            pl.BlockSpec((_NSC, _BLK, H), lambda i: (0, i, 0)),
            pl.BlockSpec((_BLK, H), lambda i: (i, 0)),
            pl.BlockSpec((H, 2 * H), lambda i: (0, 0)),
            pl.BlockSpec((1, 2 * H), lambda i: (0, 0)),
            pl.BlockSpec((1, 2 * H), lambda i: (0, 0)),
            pl.BlockSpec((1, 2 * H), lambda i: (0, 0)),
            pl.BlockSpec((2 * H, H), lambda i: (0, 0)),
            pl.BlockSpec((1, H), lambda i: (0, 0)),
        ],
        out_specs=pl.BlockSpec((_BLK, H), lambda i: (i, 0)),
        out_shape=jax.ShapeDtypeStruct((N, H), jnp.float32),
    )(h2, parts, res, W1l, b1l.reshape(1, -1), s1l.reshape(1, -1),
      bb1l.reshape(1, -1), W2l, b2l.reshape(1, -1))


def _layernorm(x, s, b):
    mu = jnp.mean(x, axis=-1, keepdims=True)
    var = jnp.var(x, axis=-1, keepdims=True)
    return (x - mu) / jnp.sqrt(var + 1e-5) * s + b


def kernel(x, edge_index, edge_attr, batch, W_enc, b_enc, edge_table, W1, b1,
           ln1s, ln1b, W2, b2, norm_s, norm_b, W_pred, b_pred):
    src = edge_index[0]
    dst = edge_index[1]

    total = _NGT * _NBUF * _CSZ
    cidx = edge_attr.astype(jnp.int32) * N + src
    gidx = jnp.concatenate(
        [cidx, jnp.zeros((total - E,), jnp.int32)]).reshape(_NGT, _NBUF, _CSZ)
    didx = jnp.concatenate(
        [dst, jnp.full((total - E,), N, jnp.int32)]).reshape(_NGT, _NBUF, _CSZ)
    zeros = jnp.zeros((_NPAD, H), jnp.float32)

    h = _encode(x, W_enc, b_enc)
    for l in range(L):
        if l == 0:
            h2 = h
            res = jnp.zeros((N, H), jnp.float32)
        else:
            h2 = jax.nn.relu(_layernorm(h, norm_s[l - 1], norm_b[l - 1]))
            res = h
        xp = _xprime(h2, edge_table)
        parts = _mp_call(xp, gidx, didx, zeros)
        h = _mlp(h2, parts, res, W1[l], b1[l], ln1s[l], ln1b[l], W2[l], b2[l])

    hf = _layernorm(h, norm_s[L - 1], norm_b[L - 1])
    sums = jax.ops.segment_sum(hf, batch, num_segments=G)
    counts = jax.ops.segment_sum(jnp.ones((N,), jnp.float32), batch,
                                 num_segments=G)
    hg = sums / jnp.maximum(counts, 1.0)[:, None]
    out = jax.nn.sigmoid(hg @ W_pred + b_pred)
    return out.reshape(-1)


# speedup vs baseline: 5.6864x; 1.8215x over previous
import functools

import jax
import jax.numpy as jnp
from jax import lax
from jax.experimental import pallas as pl
from jax.experimental.pallas import tpu as pltpu
from jax.experimental.pallas import tpu_sc as plsc

N = 10000
E = 320000
H = 128
L = 7
G = 64
VOCAB = 8
EPS = 1e-7

_BLK = 1000

_NSC = 2
_NSUB = 16
_NW = _NSC * _NSUB
_CSZ = 112
_CH = 90
_NBUF = 3
_NGRP = _CH // _NBUF
_NGT = _NW * _NGRP + 1
_NPAD = 10112
_RPT = _NPAD // _NSUB


def _mp_body(xp_hbm, gidx_hbm, didx_hbm, zeros_hbm, out_hbm,
             m_sh, gib0, gib1, dib0, dib1, r0, r1, r2,
             gs0, gs1, gs2, ss0, ss1, ss2, is0, is1):
    c = lax.axis_index("c")
    s = lax.axis_index("s")
    wid = c * _NSUB + s
    gbase = wid * _NGRP
    bufs = (r0, r1, r2)
    gsems = (gs0, gs1, gs2)
    ssems = (ss0, ss1, ss2)
    gibs = (gib0, gib1)
    dibs = (dib0, dib1)
    isems = (is0, is1)

    def _idx_load(g, p):
        pltpu.async_copy(gidx_hbm.at[gbase + g], gibs[p], isems[p])
        pltpu.async_copy(didx_hbm.at[gbase + g], dibs[p], isems[p])

    def _idx_wait(p):
        pltpu.make_async_copy(gidx_hbm.at[0], gibs[p], isems[p]).wait()
        pltpu.make_async_copy(didx_hbm.at[0], dibs[p], isems[p]).wait()

    def _start_gather(p, b):
        pltpu.async_copy(xp_hbm.at[gibs[p].at[b]], bufs[b], gsems[b])

    def _wait_gather(p, b):
        pltpu.make_async_copy(xp_hbm.at[gibs[p].at[b]], bufs[b],
                              gsems[b]).wait()

    def _start_scatter(p, b):
        pltpu.async_copy(bufs[b], m_sh.at[dibs[p].at[b]], ssems[b], add=True)

    def _wait_scatter(p, b):
        pltpu.make_async_copy(bufs[b], m_sh.at[dibs[p].at[b]],
                              ssems[b]).wait()

    pltpu.sync_copy(zeros_hbm.at[pl.ds(s * _RPT, _RPT)],
                    m_sh.at[pl.ds(s * _RPT, _RPT)])

    _idx_load(0, 0)
    _idx_load(1, 1)
    plsc.subcore_barrier()
    _idx_wait(0)
    for b in range(_NBUF):
        _start_gather(0, b)

    def _do_group(p, q, prefetch_g):
        for b in range(_NBUF):
            _wait_gather(p, b)
            _start_scatter(p, b)
        _idx_wait(q)
        for b in range(_NBUF):
            _wait_scatter(p, b)
            _start_gather(q, b)
        _idx_load(prefetch_g, p)

    def _pair(i, carry):
        g = 2 * i
        _do_group(0, 1, g + 2)
        _do_group(1, 0, g + 3)
        return carry

    lax.fori_loop(0, _NGRP // 2 - 1, _pair, 0)
    _do_group(0, 1, _NGRP)
    for b in range(_NBUF):
        _wait_gather(1, b)
        _start_scatter(1, b)
    for b in range(_NBUF):
        _wait_scatter(1, b)
    _idx_wait(0)

    plsc.subcore_barrier()
    pltpu.sync_copy(m_sh.at[pl.ds(s * _RPT, _RPT)],
                    out_hbm.at[c, pl.ds(s * _RPT, _RPT)])


_mp_call = pl.kernel(
    _mp_body,
    out_type=jax.ShapeDtypeStruct((_NSC, _NPAD, H), jnp.float32),
    mesh=plsc.VectorSubcoreMesh(core_axis_name="c", subcore_axis_name="s"),
    scratch_types=[
        pltpu.VMEM_SHARED((_NPAD, H), jnp.float32),
        pltpu.VMEM((_NBUF, _CSZ), jnp.int32),
        pltpu.VMEM((_NBUF, _CSZ), jnp.int32),
        pltpu.VMEM((_NBUF, _CSZ), jnp.int32),
        pltpu.VMEM((_NBUF, _CSZ), jnp.int32),
        pltpu.VMEM((_CSZ, H), jnp.float32),
        pltpu.VMEM((_CSZ, H), jnp.float32),
        pltpu.VMEM((_CSZ, H), jnp.float32),
        pltpu.SemaphoreType.DMA,
        pltpu.SemaphoreType.DMA,
        pltpu.SemaphoreType.DMA,
        pltpu.SemaphoreType.DMA,
        pltpu.SemaphoreType.DMA,
        pltpu.SemaphoreType.DMA,
        pltpu.SemaphoreType.DMA,
        pltpu.SemaphoreType.DMA,
    ],
)


def _enc_body(x_ref, w_ref, b_ref, out_ref):
    out_ref[...] = x_ref[...] @ w_ref[...] + b_ref[...]


def _encode(x, W_enc, b_enc):
    return pl.pallas_call(
        _enc_body,
        grid=(N // _BLK,),
        in_specs=[
            pl.BlockSpec((_BLK, H), lambda i: (i, 0)),
            pl.BlockSpec((H, H), lambda i: (0, 0)),
            pl.BlockSpec((1, H), lambda i: (0, 0)),
        ],
        out_specs=pl.BlockSpec((_BLK, H), lambda i: (i, 0)),
        out_shape=jax.ShapeDtypeStruct((N, H), jnp.float32),
    )(x, W_enc, b_enc.reshape(1, H))


def _xp_body(h2_ref, table_ref, out_ref):
    h2 = h2_ref[...]
    for a in range(VOCAB):
        out_ref[a] = jnp.maximum(h2 + table_ref[a], 0.0) + EPS


def _xprime(h2, table):
    out = pl.pallas_call(
        _xp_body,
        grid=(N // _BLK,),
        in_specs=[
            pl.BlockSpec((_BLK, H), lambda i: (i, 0)),
            pl.BlockSpec((VOCAB, H), lambda i: (0, 0)),
        ],
        out_specs=pl.BlockSpec((VOCAB, _BLK, H), lambda i: (0, i, 0)),
        out_shape=jax.ShapeDtypeStruct((VOCAB, N, H), jnp.float32),
    )(h2, table)
    return out.reshape(VOCAB * N, H)


def _mlp_body(h2_ref, parts_ref, res_ref, w1_ref, b1_ref, s1_ref, bb1_ref,
              w2_ref, b2_ref, out_ref):
    hsum = h2_ref[...] + parts_ref[0] + parts_ref[1]
    t = hsum @ w1_ref[...] + b1_ref[...]
    mu = jnp.mean(t, axis=-1, keepdims=True)
    var = jnp.mean((t - mu) ** 2, axis=-1, keepdims=True)
    t = (t - mu) / jnp.sqrt(var + 1e-5) * s1_ref[...] + bb1_ref[...]
    t = jnp.maximum(t, 0.0)
    out_ref[...] = t @ w2_ref[...] + b2_ref[...] + res_ref[...]


def _mlp(h2, parts, res, W1l, b1l, s1l, bb1l, W2l, b2l):
    return pl.pallas_call(
        _mlp_body,
        grid=(N // _BLK,),
        in_specs=[
            pl.BlockSpec((_BLK, H), lambda i: (i, 0)),
            pl.BlockSpec((_NSC, _BLK, H), lambda i: (0, i, 0)),
            pl.BlockSpec((_BLK, H), lambda i: (i, 0)),
            pl.BlockSpec((H, 2 * H), lambda i: (0, 0)),
            pl.BlockSpec((1, 2 * H), lambda i: (0, 0)),
            pl.BlockSpec((1, 2 * H), lambda i: (0, 0)),
            pl.BlockSpec((1, 2 * H), lambda i: (0, 0)),
            pl.BlockSpec((2 * H, H), lambda i: (0, 0)),
            pl.BlockSpec((1, H), lambda i: (0, 0)),
        ],
        out_specs=pl.BlockSpec((_BLK, H), lambda i: (i, 0)),
        out_shape=jax.ShapeDtypeStruct((N, H), jnp.float32),
    )(h2, parts, res, W1l, b1l.reshape(1, -1), s1l.reshape(1, -1),
      bb1l.reshape(1, -1), W2l, b2l.reshape(1, -1))


def _layernorm(x, s, b):
    mu = jnp.mean(x, axis=-1, keepdims=True)
    var = jnp.var(x, axis=-1, keepdims=True)
    return (x - mu) / jnp.sqrt(var + 1e-5) * s + b


def kernel(x, edge_index, edge_attr, batch, W_enc, b_enc, edge_table, W1, b1,
           ln1s, ln1b, W2, b2, norm_s, norm_b, W_pred, b_pred):
    src = edge_index[0]
    dst = edge_index[1]

    total = _NGT * _NBUF * _CSZ
    cidx = edge_attr.astype(jnp.int32) * N + src
    gidx = jnp.concatenate(
        [cidx, jnp.zeros((total - E,), jnp.int32)]).reshape(_NGT, _NBUF, _CSZ)
    didx = jnp.concatenate(
        [dst, jnp.full((total - E,), N, jnp.int32)]).reshape(_NGT, _NBUF, _CSZ)
    zeros = jnp.zeros((_NPAD, H), jnp.float32)

    h = _encode(x, W_enc, b_enc)
    for l in range(L):
        if l == 0:
            h2 = h
            res = jnp.zeros((N, H), jnp.float32)
        else:
            h2 = jax.nn.relu(_layernorm(h, norm_s[l - 1], norm_b[l - 1]))
            res = h
        xp = _xprime(h2, edge_table)
        parts = _mp_call(xp, gidx, didx, zeros)
        h = _mlp(h2, parts, res, W1[l], b1[l], ln1s[l], ln1b[l], W2[l], b2[l])

    hf = _layernorm(h, norm_s[L - 1], norm_b[L - 1])
    sums = jax.ops.segment_sum(hf, batch, num_segments=G)
    counts = jax.ops.segment_sum(jnp.ones((N,), jnp.float32), batch,
                                 num_segments=G)
    hg = sums / jnp.maximum(counts, 1.0)[:, None]
    out = jax.nn.sigmoid(hg @ W_pred + b_pred)
    return out.reshape(-1)
